# baseline (speedup 1.0000x reference)
import functools

import jax
import jax.numpy as jnp
from jax import lax
from jax.experimental import pallas as pl
from jax.experimental.pallas import tpu as pltpu
from jax.experimental.pallas import tpu_sc as plsc

_NC, _NS = 2, 16
_WROWS = 24
_TROWS = 32
_NB = 2
_L = 16


def _ring(nrows, pos_in, x_in, x_out, compute, n_batches):
    pos_in().start()
    for k in range(_NB):
        x_in(k, k).start()
    pos_in().wait()

    def body(b, carry):
        slot = lax.rem(b, _NB)
        x_in(b, slot).wait()

        @pl.when(b >= _NB)
        def _():
            x_out(b - _NB, slot).wait()

        compute(slot, nrows)

        x_out(b, slot).start()

        @pl.when(b + _NB < n_batches)
        def _():
            x_in(b + _NB, slot).start()

        return carry

    lax.fori_loop(0, n_batches, body, jnp.int32(0))

    for b in range(n_batches - _NB, n_batches):
        x_out(b, b % _NB).wait()


def _sc_body(x_hbm, pos_hbm, o_hbm, posb, xb, ob, idxb, psem, xsem, osem):
    B, S, D = x_hbm.shape
    n_full = S // _WROWS
    tail_start = S - _TROWS
    wid = lax.axis_index("s") * _NC + lax.axis_index("c")

    def compute(slot, nrows):
        @plsc.parallel_loop(0, nrows, 1)
        def _(r):
            for j in range(D // _L):
                ob[slot, r, pl.ds(j * _L, _L)] = (
                    xb[slot, r, pl.ds(j * _L, _L)] + posb[r, pl.ds(j * _L, _L)])

    @pl.when(wid < n_full)
    def _():
        start = pl.multiple_of(wid * _WROWS, 8)

        def pos_in():
            return pltpu.make_async_copy(
                pos_hbm.at[pl.ds(start, _WROWS)],
                posb.at[pl.ds(0, _WROWS)], psem)

        def x_in(b, slot):
            return pltpu.make_async_copy(
                x_hbm.at[b, pl.ds(start, _WROWS)],
                xb.at[slot, pl.ds(0, _WROWS)], xsem.at[slot])

        def x_out(b, slot):
            return pltpu.make_async_copy(
                ob.at[slot, pl.ds(0, _WROWS)],
                o_hbm.at[b, pl.ds(start, _WROWS)], osem.at[slot])

        _ring(_WROWS, pos_in, x_in, x_out, compute, B)

    @pl.when(wid == n_full)
    def _():
        iota = lax.iota(jnp.int32, _L)
        idxb[pl.ds(0, _L)] = tail_start + iota
        idxb[pl.ds(_L, _L)] = tail_start + _L + iota

        def pos_in():
            return pltpu.make_async_copy(pos_hbm.at[idxb], posb, psem)

        def x_in(b, slot):
            return pltpu.make_async_copy(
                x_hbm.at[b].at[idxb], xb.at[slot], xsem.at[slot])

        def x_out(b, slot):
            return pltpu.make_async_copy(
                ob.at[slot], o_hbm.at[b].at[idxb], osem.at[slot])

        _ring(_TROWS, pos_in, x_in, x_out, compute, B)


def kernel(x, pos_table):
    B, S, D = x.shape
    run = functools.partial(
        pl.kernel,
        out_type=jax.ShapeDtypeStruct((B, S, D), x.dtype),
        mesh=plsc.VectorSubcoreMesh(core_axis_name="c", subcore_axis_name="s"),
        scratch_types=[
            pltpu.VMEM((_TROWS, D), x.dtype),
            pltpu.VMEM((_NB, _TROWS, D), x.dtype),
            pltpu.VMEM((_NB, _TROWS, D), x.dtype),
            pltpu.VMEM((_TROWS,), jnp.int32),
            pltpu.SemaphoreType.DMA,
            pltpu.SemaphoreType.DMA((_NB,)),
            pltpu.SemaphoreType.DMA((_NB,)),
        ],
    )(_sc_body)
    return run(x, pos_table)


# speedup vs baseline: 1.0962x; 1.0962x over previous
import functools

import jax
import jax.numpy as jnp
from jax import lax
from jax.experimental import pallas as pl
from jax.experimental.pallas import tpu as pltpu
from jax.experimental.pallas import tpu_sc as plsc

_NC, _NS = 2, 16
_WROWS = 24
_TROWS = 24
_NB = 3
_L = 16


def _ring(nrows, pos_in, x_in, x_out, compute, n_batches):
    pos_in().start()
    for k in range(_NB):
        x_in(k, k).start()
    pos_in().wait()

    def body(b, carry):
        slot = lax.rem(b, _NB)
        x_in(b, slot).wait()

        @pl.when(b >= _NB)
        def _():
            x_out(b - _NB, slot).wait()

        compute(slot, nrows)

        x_out(b, slot).start()

        @pl.when(b + _NB < n_batches)
        def _():
            x_in(b + _NB, slot).start()

        return carry

    lax.fori_loop(0, n_batches, body, jnp.int32(0))

    for b in range(n_batches - _NB, n_batches):
        x_out(b, b % _NB).wait()


def _sc_body(x_hbm, pos_hbm, o_hbm, posb, xb, ob, idxb, psem, xsem, osem):
    B, S, D = x_hbm.shape
    n_full = S // _WROWS
    tail_start = S - _TROWS
    wid = lax.axis_index("s") * _NC + lax.axis_index("c")

    def compute(slot, nrows):
        @plsc.parallel_loop(0, nrows, 1)
        def _(r):
            for j in range(D // _L):
                ob[slot, r, pl.ds(j * _L, _L)] = (
                    xb[slot, r, pl.ds(j * _L, _L)] + posb[r, pl.ds(j * _L, _L)])

    @pl.when(wid < n_full)
    def _():
        start = pl.multiple_of(wid * _WROWS, 8)

        def pos_in():
            return pltpu.make_async_copy(
                pos_hbm.at[pl.ds(start, _WROWS)],
                posb.at[pl.ds(0, _WROWS)], psem)

        def x_in(b, slot):
            return pltpu.make_async_copy(
                x_hbm.at[b, pl.ds(start, _WROWS)],
                xb.at[slot, pl.ds(0, _WROWS)], xsem.at[slot])

        def x_out(b, slot):
            return pltpu.make_async_copy(
                ob.at[slot, pl.ds(0, _WROWS)],
                o_hbm.at[b, pl.ds(start, _WROWS)], osem.at[slot])

        _ring(_WROWS, pos_in, x_in, x_out, compute, B)

    @pl.when(wid == n_full)
    def _():
        iota = lax.iota(jnp.int32, _L)
        idxb[pl.ds(0, _L)] = tail_start + iota
        idxb[pl.ds(_TROWS - _L, _L)] = tail_start + (_TROWS - _L) + iota

        def pos_in():
            return pltpu.make_async_copy(pos_hbm.at[idxb], posb, psem)

        def x_in(b, slot):
            return pltpu.make_async_copy(
                x_hbm.at[b].at[idxb], xb.at[slot], xsem.at[slot])

        def x_out(b, slot):
            return pltpu.make_async_copy(
                ob.at[slot], o_hbm.at[b].at[idxb], osem.at[slot])

        _ring(_TROWS, pos_in, x_in, x_out, compute, B)


def kernel(x, pos_table):
    B, S, D = x.shape
    run = functools.partial(
        pl.kernel,
        out_type=jax.ShapeDtypeStruct((B, S, D), x.dtype),
        mesh=plsc.VectorSubcoreMesh(core_axis_name="c", subcore_axis_name="s"),
        scratch_types=[
            pltpu.VMEM((_TROWS, D), x.dtype),
            pltpu.VMEM((_NB, _TROWS, D), x.dtype),
            pltpu.VMEM((_NB, _TROWS, D), x.dtype),
            pltpu.VMEM((_TROWS,), jnp.int32),
            pltpu.SemaphoreType.DMA,
            pltpu.SemaphoreType.DMA((_NB,)),
            pltpu.SemaphoreType.DMA((_NB,)),
        ],
    )(_sc_body)
    return run(x, pos_table)


# speedup vs baseline: 1.1766x; 1.0733x over previous
import functools

import jax
import jax.numpy as jnp
import numpy as np
from jax import lax
from jax.experimental import pallas as pl
from jax.experimental.pallas import tpu as pltpu
from jax.experimental.pallas import tpu_sc as plsc

_NC, _NS = 2, 16
_NW = _NC * _NS
_WROWS = 18
_NB = 3
_L = 16


def _sc_body(x_hbm, pos_hbm, idx_hbm, o_hbm, posb, xb, ob, idxb,
             psem, xsem, osem, isem):
    B, S, D = x_hbm.shape
    wid = lax.axis_index("s") * _NC + lax.axis_index("c")

    pltpu.make_async_copy(idx_hbm.at[wid], idxb, isem).start()
    pltpu.make_async_copy(idx_hbm.at[wid], idxb, isem).wait()

    def pos_in():
        return pltpu.make_async_copy(pos_hbm.at[idxb], posb, psem)

    def x_in(b, slot):
        return pltpu.make_async_copy(
            x_hbm.at[b].at[idxb], xb.at[slot], xsem.at[slot])

    def x_out(b, slot):
        return pltpu.make_async_copy(
            ob.at[slot], o_hbm.at[b].at[idxb], osem.at[slot])

    pos_in().start()
    for k in range(_NB):
        x_in(k, k).start()
    pos_in().wait()

    def body(b, carry):
        slot = lax.rem(b, _NB)
        x_in(b, slot).wait()

        @pl.when(b >= _NB)
        def _():
            x_out(b - _NB, slot).wait()

        @plsc.parallel_loop(0, _WROWS, 1)
        def _(r):
            for j in range(D // _L):
                ob[slot, r, pl.ds(j * _L, _L)] = (
                    xb[slot, r, pl.ds(j * _L, _L)] + posb[r, pl.ds(j * _L, _L)])

        x_out(b, slot).start()

        @pl.when(b + _NB < B)
        def _():
            x_in(b + _NB, slot).start()

        return carry

    lax.fori_loop(0, B, body, jnp.int32(0))

    for b in range(B - _NB, B):
        x_out(b, b % _NB).wait()


def kernel(x, pos_table):
    B, S, D = x.shape
    starts = np.minimum(np.arange(_NW) * _WROWS, S - _WROWS)
    idx_all = jnp.asarray(
        starts[:, None] + np.arange(_WROWS)[None, :], dtype=jnp.int32)
    run = functools.partial(
        pl.kernel,
        out_type=jax.ShapeDtypeStruct((B, S, D), x.dtype),
        mesh=plsc.VectorSubcoreMesh(core_axis_name="c", subcore_axis_name="s"),
        scratch_types=[
            pltpu.VMEM((_WROWS, D), x.dtype),
            pltpu.VMEM((_NB, _WROWS, D), x.dtype),
            pltpu.VMEM((_NB, _WROWS, D), x.dtype),
            pltpu.VMEM((_WROWS,), jnp.int32),
            pltpu.SemaphoreType.DMA,
            pltpu.SemaphoreType.DMA((_NB,)),
            pltpu.SemaphoreType.DMA((_NB,)),
            pltpu.SemaphoreType.DMA,
        ],
    )(_sc_body)
    return run(x, pos_table, idx_all)
